# Initial kernel scaffold; baseline (speedup 1.0000x reference)
import jax
import jax.numpy as jnp
from jax.experimental import pallas as pl


def kernel(x, edge_index, W1, b1, W2, b2):
    raise NotImplementedError("write your pallas kernel here")


# speedup vs baseline: 15.8272x; 15.8272x over previous
import functools

import jax
import jax.numpy as jnp
from jax import lax
from jax.experimental import pallas as pl
from jax.experimental.pallas import tpu as pltpu
from jax.experimental.pallas import tpu_sc as plsc

NNODES = 10000
D = 128
NC = 2
NS = 16
NTILES = NC * NS
K = 128
NPAD = 10112
RPT = NPAD // NS
RPB = 632
NBLK = NPAD // RPB


def _sc_mesh():
    return plsc.VectorSubcoreMesh(core_axis_name="c", subcore_axis_name="s")


def _row_chunks(total, step):
    out = []
    q0 = 0
    while q0 < total:
        out.append((q0, min(step, total - q0)))
        q0 += step
    return out


@functools.partial(jax.jit, static_argnums=(1,))
def _deg_call(dst, epad):
    ept = epad // NTILES
    nchunk = ept // K

    @functools.partial(
        pl.kernel,
        out_type=jax.ShapeDtypeStruct((NC * NPAD,), jnp.float32),
        mesh=_sc_mesh(),
        scratch_types=[
            pltpu.VMEM((K,), jnp.int32),
            pltpu.VMEM((K,), jnp.float32),
            pltpu.VMEM((RPT,), jnp.float32),
            pltpu.VMEM_SHARED((NPAD,), jnp.float32),
        ],
    )
    def deg_kernel(dst_hbm, zrow_hbm, ones_hbm, out_hbm, idx_v, ones_v,
                   stage_v, acc_sh):
        c = lax.axis_index("c")
        s = lax.axis_index("s")
        pltpu.sync_copy(ones_hbm, ones_v)
        pltpu.sync_copy(zrow_hbm, stage_v)
        pltpu.sync_copy(stage_v, acc_sh.at[pl.ds(s * RPT, RPT)])
        plsc.subcore_barrier()
        base = (c * NS + s) * ept

        def body(j, carry):
            off = pl.multiple_of(base + j * K, 8)
            pltpu.sync_copy(dst_hbm.at[pl.ds(off, K)], idx_v)
            pltpu.sync_copy(ones_v, acc_sh.at[idx_v], add=True)
            return carry

        lax.fori_loop(0, nchunk, body, 0)
        plsc.subcore_barrier()
        pltpu.sync_copy(acc_sh.at[pl.ds(s * RPT, RPT)], stage_v)
        pltpu.sync_copy(stage_v, out_hbm.at[pl.ds(c * NPAD + s * RPT, RPT)])

    zrow = jnp.zeros((RPT,), jnp.float32)
    ones = jnp.ones((K,), jnp.float32)
    return deg_kernel(dst, zrow, ones)


@functools.partial(jax.jit, static_argnums=(3,))
def _agg_call(hp, src, dst, epad):
    ept = epad // NTILES
    nchunk = ept // K

    @functools.partial(
        pl.kernel,
        out_type=jax.ShapeDtypeStruct((NC * NPAD, D), jnp.float32),
        mesh=_sc_mesh(),
        scratch_types=[
            pltpu.VMEM((K,), jnp.int32),
            pltpu.VMEM((K,), jnp.int32),
            pltpu.VMEM((K, D), jnp.float32),
            pltpu.VMEM_SHARED((NPAD, D), jnp.float32),
            pltpu.SemaphoreType.DMA,
        ],
    )
    def agg_kernel(hp_hbm, src_hbm, dst_hbm, zrows_hbm, out_hbm,
                   sidx_v, didx_v, rows_v, acc_sh, sem):
        c = lax.axis_index("c")
        s = lax.axis_index("s")
        r0 = s * RPT
        pltpu.sync_copy(zrows_hbm, rows_v)
        for q0, qn in _row_chunks(RPT, K):
            pltpu.sync_copy(rows_v.at[pl.ds(0, qn)],
                            acc_sh.at[pl.ds(r0 + q0, qn), :])
        plsc.subcore_barrier()
        base = (c * NS + s) * ept

        def body(j, carry):
            off = pl.multiple_of(base + j * K, 8)
            pltpu.sync_copy(src_hbm.at[pl.ds(off, K)], sidx_v)
            pltpu.sync_copy(dst_hbm.at[pl.ds(off, K)], didx_v)
            pltpu.async_copy(hp_hbm.at[sidx_v], rows_v, sem).wait()
            pltpu.sync_copy(rows_v, acc_sh.at[didx_v], add=True)
            return carry

        lax.fori_loop(0, nchunk, body, 0)
        plsc.subcore_barrier()
        for q0, qn in _row_chunks(RPT, K):
            pltpu.sync_copy(acc_sh.at[pl.ds(r0 + q0, qn), :],
                            rows_v.at[pl.ds(0, qn)])
            pltpu.sync_copy(rows_v.at[pl.ds(0, qn)],
                            out_hbm.at[pl.ds(c * NPAD + r0 + q0, qn), :])

    zrows = jnp.zeros((K, D), jnp.float32)
    return agg_kernel(hp, src, dst, zrows)


def _tc1_body(x_ref, w_ref, dinv_ref, out_ref):
    h = jnp.dot(x_ref[...], w_ref[...], preferred_element_type=jnp.float32)
    out_ref[...] = h * dinv_ref[...]


def _tc1(xp, w1, dinv_col):
    return pl.pallas_call(
        _tc1_body,
        grid=(NBLK,),
        in_specs=[
            pl.BlockSpec((RPB, D), lambda i: (i, 0)),
            pl.BlockSpec((D, D), lambda i: (0, 0)),
            pl.BlockSpec((RPB, 1), lambda i: (i, 0)),
        ],
        out_specs=pl.BlockSpec((RPB, D), lambda i: (i, 0)),
        out_shape=jax.ShapeDtypeStruct((NPAD, D), jnp.float32),
    )(xp, w1, dinv_col)


def _tc2_body(a0_ref, a1_ref, hp_ref, dinv_ref, b_ref, w_ref, out_ref):
    pre = dinv_ref[...] * (a0_ref[...] + a1_ref[...] + hp_ref[...]) + b_ref[...]
    x2 = jnp.maximum(pre, 0.0)
    h = jnp.dot(x2, w_ref[...], preferred_element_type=jnp.float32)
    out_ref[...] = h * dinv_ref[...]


def _tc2(g1, h1p, dinv_col, b1r, w2):
    return pl.pallas_call(
        _tc2_body,
        grid=(NBLK,),
        in_specs=[
            pl.BlockSpec((RPB, D), lambda i: (i, 0)),
            pl.BlockSpec((RPB, D), lambda i: (i + NBLK, 0)),
            pl.BlockSpec((RPB, D), lambda i: (i, 0)),
            pl.BlockSpec((RPB, 1), lambda i: (i, 0)),
            pl.BlockSpec((1, D), lambda i: (0, 0)),
            pl.BlockSpec((D, D), lambda i: (0, 0)),
        ],
        out_specs=pl.BlockSpec((RPB, D), lambda i: (i, 0)),
        out_shape=jax.ShapeDtypeStruct((NPAD, D), jnp.float32),
    )(g1, g1, h1p, dinv_col, b1r, w2)


def _tc3_body(a0_ref, a1_ref, hp_ref, dinv_ref, b_ref, out_ref):
    pre = dinv_ref[...] * (a0_ref[...] + a1_ref[...] + hp_ref[...]) + b_ref[...]
    out_ref[...] = jnp.maximum(pre, 0.0)


def _tc3(g2, h2p, dinv_col, b2r):
    return pl.pallas_call(
        _tc3_body,
        grid=(NBLK,),
        in_specs=[
            pl.BlockSpec((RPB, D), lambda i: (i, 0)),
            pl.BlockSpec((RPB, D), lambda i: (i + NBLK, 0)),
            pl.BlockSpec((RPB, D), lambda i: (i, 0)),
            pl.BlockSpec((RPB, 1), lambda i: (i, 0)),
            pl.BlockSpec((1, D), lambda i: (0, 0)),
        ],
        out_specs=pl.BlockSpec((RPB, D), lambda i: (i, 0)),
        out_shape=jax.ShapeDtypeStruct((NPAD, D), jnp.float32),
    )(g2, g2, h2p, dinv_col, b2r)


def kernel(x, edge_index, W1, b1, W2, b2):
    e = edge_index.shape[1]
    epad = -(-e // (NTILES * K)) * (NTILES * K)
    pad = epad - e
    src = edge_index[0]
    dst = edge_index[1]
    if pad:
        ar = jnp.arange(pad, dtype=jnp.int32)
        src = jnp.concatenate([src, ar % NNODES])
        dst = jnp.concatenate([dst, NNODES + ar % (NPAD - NNODES)])
    xp = jnp.zeros((NPAD, D), jnp.float32).at[:NNODES].set(x)

    deg2 = _deg_call(dst, epad)
    degsum = deg2[:NPAD] + deg2[NPAD:] + 1.0
    dinv_col = lax.rsqrt(degsum)[:, None]

    b1r = b1.reshape(1, D)
    b2r = b2.reshape(1, D)

    h1p = _tc1(xp, W1, dinv_col)
    g1 = _agg_call(h1p, src, dst, epad)
    h2p = _tc2(g1, h1p, dinv_col, b1r, W2)
    g2 = _agg_call(h2p, src, dst, epad)
    out_full = _tc3(g2, h2p, dinv_col, b2r)
    return out_full[:NNODES]


# speedup vs baseline: 29.3308x; 1.8532x over previous
import functools

import jax
import jax.numpy as jnp
from jax import lax
from jax.experimental import pallas as pl
from jax.experimental.pallas import tpu as pltpu
from jax.experimental.pallas import tpu_sc as plsc

NNODES = 10000
D = 128
NC = 2
NS = 16
NTILES = NC * NS
CH = 80
NPAD = 10112
RPT = NPAD // NS
RPB = 632
NBLK = NPAD // RPB


def _sc_mesh():
    return plsc.VectorSubcoreMesh(core_axis_name="c", subcore_axis_name="s")


def _row_chunks(total, step):
    out = []
    q0 = 0
    while q0 < total:
        out.append((q0, min(step, total - q0)))
        q0 += step
    return out


@functools.partial(jax.jit, static_argnums=(1, 2))
def _deg_call(dst2d, ncht, nrow):
    @functools.partial(
        pl.kernel,
        out_type=jax.ShapeDtypeStruct((NC * NPAD,), jnp.float32),
        mesh=_sc_mesh(),
        scratch_types=[
            pltpu.VMEM((nrow, CH), jnp.int32),
            pltpu.VMEM((CH,), jnp.float32),
            pltpu.VMEM((RPT,), jnp.float32),
            pltpu.VMEM_SHARED((NPAD,), jnp.float32),
        ],
    )
    def deg_kernel(dst_hbm, zrow_hbm, ones_hbm, out_hbm, didx, ones_v,
                   stage_v, acc_sh):
        c = lax.axis_index("c")
        s = lax.axis_index("s")
        w = c * NS + s
        pltpu.sync_copy(ones_hbm, ones_v)
        pltpu.sync_copy(zrow_hbm, stage_v)
        pltpu.sync_copy(stage_v, acc_sh.at[pl.ds(s * RPT, RPT)])
        pltpu.sync_copy(dst_hbm.at[pl.ds(w * nrow, nrow)], didx)
        plsc.subcore_barrier()

        def body(j, carry):
            pltpu.sync_copy(ones_v, acc_sh.at[didx.at[j]], add=True)
            return carry

        lax.fori_loop(0, ncht, body, 0)
        plsc.subcore_barrier()
        pltpu.sync_copy(acc_sh.at[pl.ds(s * RPT, RPT)], stage_v)
        pltpu.sync_copy(stage_v, out_hbm.at[pl.ds(c * NPAD + s * RPT, RPT)])

    zrow = jnp.zeros((RPT,), jnp.float32)
    ones = jnp.ones((CH,), jnp.float32)
    return deg_kernel(dst2d, zrow, ones)


@functools.partial(jax.jit, static_argnums=(3, 4))
def _agg_call(hp, src_flat, dst2d, ncht, nrow):
    wb_chunks = _row_chunks(RPT, CH)

    @functools.partial(
        pl.kernel,
        out_type=jax.ShapeDtypeStruct((NC * NPAD, D), jnp.float32),
        mesh=_sc_mesh(),
        scratch_types=[
            pltpu.VMEM((nrow * CH,), jnp.int32),
            pltpu.VMEM((nrow, CH), jnp.int32),
            pltpu.VMEM((2, CH, D), jnp.float32),
            pltpu.VMEM_SHARED((NPAD, D), jnp.float32),
            pltpu.SemaphoreType.DMA,
            pltpu.SemaphoreType.DMA,
        ],
    )
    def agg_kernel(hp_hbm, src_hbm, dst_hbm, zrows_hbm, out_hbm,
                   sidx, didx, rows, acc_sh, sem0, sem1):
        c = lax.axis_index("c")
        s = lax.axis_index("s")
        w = c * NS + s
        r0 = s * RPT
        sems = (sem0, sem1)

        pltpu.sync_copy(zrows_hbm, rows.at[0])
        for q0, qn in wb_chunks:
            pltpu.sync_copy(rows.at[0, pl.ds(0, qn)],
                            acc_sh.at[pl.ds(r0 + q0, qn), :])
        pltpu.sync_copy(src_hbm.at[pl.ds(w * (nrow * CH), nrow * CH)], sidx)
        pltpu.sync_copy(dst_hbm.at[pl.ds(w * nrow, nrow)], didx)
        plsc.subcore_barrier()

        def gather(j, b):
            return pltpu.make_async_copy(hp_hbm.at[sidx.at[pl.ds(j * CH, CH)]],
                                         rows.at[b], sems[b])

        gather(0, 0).start()
        gather(1, 1).start()

        def body(jj, carry):
            for b in (0, 1):
                j = jj * 2 + b

                @pl.when(j < ncht)
                def _process():
                    gather(j, b).wait()
                    pltpu.sync_copy(rows.at[b], acc_sh.at[didx.at[j]],
                                    add=True)

                    @pl.when(j + 2 < ncht)
                    def _next():
                        gather(j + 2, b).start()
            return carry

        lax.fori_loop(0, (ncht + 1) // 2, body, 0)
        plsc.subcore_barrier()

        def wb(i, phase):
            q0, qn = wb_chunks[i]
            b = i % 2
            cp = pltpu.make_async_copy(
                rows.at[b, pl.ds(0, qn)],
                out_hbm.at[pl.ds(c * NPAD + r0 + q0, qn), :], sems[b])
            if phase == 0:
                pltpu.sync_copy(acc_sh.at[pl.ds(r0 + q0, qn), :],
                                rows.at[b, pl.ds(0, qn)])
                cp.start()
            else:
                cp.wait()

        for i in range(len(wb_chunks)):
            if i >= 2:
                wb(i - 2, 1)
            wb(i, 0)
        for i in range(max(0, len(wb_chunks) - 2), len(wb_chunks)):
            wb(i, 1)

    zrows = jnp.zeros((CH, D), jnp.float32)
    return agg_kernel(hp, src_flat, dst2d, zrows)


def _tc1_body(x_ref, w_ref, dinv_ref, out_ref):
    h = jnp.dot(x_ref[...], w_ref[...], preferred_element_type=jnp.float32)
    out_ref[...] = h * dinv_ref[...]


def _tc1(xp, w1, dinv_col):
    return pl.pallas_call(
        _tc1_body,
        grid=(NBLK,),
        in_specs=[
            pl.BlockSpec((RPB, D), lambda i: (i, 0)),
            pl.BlockSpec((D, D), lambda i: (0, 0)),
            pl.BlockSpec((RPB, 1), lambda i: (i, 0)),
        ],
        out_specs=pl.BlockSpec((RPB, D), lambda i: (i, 0)),
        out_shape=jax.ShapeDtypeStruct((NPAD, D), jnp.float32),
    )(xp, w1, dinv_col)


def _tc2_body(a0_ref, a1_ref, hp_ref, dinv_ref, b_ref, w_ref, out_ref):
    pre = dinv_ref[...] * (a0_ref[...] + a1_ref[...] + hp_ref[...]) + b_ref[...]
    x2 = jnp.maximum(pre, 0.0)
    h = jnp.dot(x2, w_ref[...], preferred_element_type=jnp.float32)
    out_ref[...] = h * dinv_ref[...]


def _tc2(g1, h1p, dinv_col, b1r, w2):
    return pl.pallas_call(
        _tc2_body,
        grid=(NBLK,),
        in_specs=[
            pl.BlockSpec((RPB, D), lambda i: (i, 0)),
            pl.BlockSpec((RPB, D), lambda i: (i + NBLK, 0)),
            pl.BlockSpec((RPB, D), lambda i: (i, 0)),
            pl.BlockSpec((RPB, 1), lambda i: (i, 0)),
            pl.BlockSpec((1, D), lambda i: (0, 0)),
            pl.BlockSpec((D, D), lambda i: (0, 0)),
        ],
        out_specs=pl.BlockSpec((RPB, D), lambda i: (i, 0)),
        out_shape=jax.ShapeDtypeStruct((NPAD, D), jnp.float32),
    )(g1, g1, h1p, dinv_col, b1r, w2)


def _tc3_body(a0_ref, a1_ref, hp_ref, dinv_ref, b_ref, out_ref):
    pre = dinv_ref[...] * (a0_ref[...] + a1_ref[...] + hp_ref[...]) + b_ref[...]
    out_ref[...] = jnp.maximum(pre, 0.0)


def _tc3(g2, h2p, dinv_col, b2r):
    return pl.pallas_call(
        _tc3_body,
        grid=(NBLK,),
        in_specs=[
            pl.BlockSpec((RPB, D), lambda i: (i, 0)),
            pl.BlockSpec((RPB, D), lambda i: (i + NBLK, 0)),
            pl.BlockSpec((RPB, D), lambda i: (i, 0)),
            pl.BlockSpec((RPB, 1), lambda i: (i, 0)),
            pl.BlockSpec((1, D), lambda i: (0, 0)),
        ],
        out_specs=pl.BlockSpec((RPB, D), lambda i: (i, 0)),
        out_shape=jax.ShapeDtypeStruct((NPAD, D), jnp.float32),
    )(g2, g2, h2p, dinv_col, b2r)


def kernel(x, edge_index, W1, b1, W2, b2):
    e = edge_index.shape[1]
    ept0 = -(-e // NTILES)
    ncht = -(-ept0 // CH)
    nrow = -(-ncht // 8) * 8
    epad = NTILES * ncht * CH
    pad = epad - e
    src = edge_index[0]
    dst = edge_index[1]
    if pad:
        ar = jnp.arange(pad, dtype=jnp.int32)
        src = jnp.concatenate([src, ar % NNODES])
        dst = jnp.concatenate([dst, NNODES + ar % (NPAD - NNODES)])

    def to_tiles(v):
        v3 = v.reshape(NTILES, ncht, CH)
        if nrow != ncht:
            v3 = jnp.pad(v3, ((0, 0), (0, nrow - ncht), (0, 0)))
        return v3.reshape(NTILES * nrow, CH)

    src_flat = to_tiles(src).reshape(-1)
    dst2d = to_tiles(dst)
    xp = jnp.zeros((NPAD, D), jnp.float32).at[:NNODES].set(x)

    deg2 = _deg_call(dst2d, ncht, nrow)
    degsum = deg2[:NPAD] + deg2[NPAD:] + 1.0
    dinv_col = lax.rsqrt(degsum)[:, None]

    b1r = b1.reshape(1, D)
    b2r = b2.reshape(1, D)

    h1p = _tc1(xp, W1, dinv_col)
    g1 = _agg_call(h1p, src_flat, dst2d, ncht, nrow)
    h2p = _tc2(g1, h1p, dinv_col, b1r, W2)
    g2 = _agg_call(h2p, src_flat, dst2d, ncht, nrow)
    out_full = _tc3(g2, h2p, dinv_col, b2r)
    return out_full[:NNODES]


# speedup vs baseline: 29.7990x; 1.0160x over previous
import functools

import jax
import jax.numpy as jnp
from jax import lax
from jax.experimental import pallas as pl
from jax.experimental.pallas import tpu as pltpu
from jax.experimental.pallas import tpu_sc as plsc

NNODES = 10000
D = 128
NC = 2
NS = 16
NTILES = NC * NS
CH = 80
NPAD = 10112
RPT = NPAD // NS
RPB = 632
NBLK = NPAD // RPB


def _sc_mesh():
    return plsc.VectorSubcoreMesh(core_axis_name="c", subcore_axis_name="s")


def _row_chunks(total, step):
    out = []
    q0 = 0
    while q0 < total:
        out.append((q0, min(step, total - q0)))
        q0 += step
    return out


@functools.partial(jax.jit, static_argnums=(1, 2))
def _deg_call(dst2d, ncht, nrow):
    @functools.partial(
        pl.kernel,
        out_type=jax.ShapeDtypeStruct((NC * NPAD,), jnp.float32),
        mesh=_sc_mesh(),
        scratch_types=[
            pltpu.VMEM((nrow, CH), jnp.int32),
            pltpu.VMEM((CH,), jnp.float32),
            pltpu.VMEM((RPT,), jnp.float32),
            pltpu.VMEM_SHARED((NPAD,), jnp.float32),
            pltpu.SemaphoreType.DMA,
        ],
    )
    def deg_kernel(dst_hbm, zrow_hbm, ones_hbm, out_hbm, didx, ones_v,
                   stage_v, acc_sh, dsem):
        c = lax.axis_index("c")
        s = lax.axis_index("s")
        w = c * NS + s
        pltpu.sync_copy(ones_hbm, ones_v)
        pltpu.sync_copy(zrow_hbm, stage_v)
        pltpu.sync_copy(stage_v, acc_sh.at[pl.ds(s * RPT, RPT)])
        pltpu.sync_copy(dst_hbm.at[pl.ds(w * nrow, nrow)], didx)
        plsc.subcore_barrier()

        def body(j, carry):
            pltpu.async_copy(ones_v, acc_sh.at[didx.at[j]], dsem, add=True)
            return carry

        lax.fori_loop(0, ncht, body, 0)

        def drain(j, carry):
            pltpu.make_async_copy(ones_v, acc_sh.at[didx.at[j]], dsem).wait()
            return carry

        lax.fori_loop(0, ncht, drain, 0)
        plsc.subcore_barrier()
        pltpu.sync_copy(acc_sh.at[pl.ds(s * RPT, RPT)], stage_v)
        pltpu.sync_copy(stage_v, out_hbm.at[pl.ds(c * NPAD + s * RPT, RPT)])

    zrow = jnp.zeros((RPT,), jnp.float32)
    ones = jnp.ones((CH,), jnp.float32)
    return deg_kernel(dst2d, zrow, ones)


@functools.partial(jax.jit, static_argnums=(3, 4))
def _agg_call(hp, src_flat, dst2d, ncht, nrow):
    wb_chunks = _row_chunks(RPT, CH)

    @functools.partial(
        pl.kernel,
        out_type=jax.ShapeDtypeStruct((NC * NPAD, D), jnp.float32),
        mesh=_sc_mesh(),
        scratch_types=[
            pltpu.VMEM((nrow * CH,), jnp.int32),
            pltpu.VMEM((nrow, CH), jnp.int32),
            pltpu.VMEM((2, CH, D), jnp.float32),
            pltpu.VMEM_SHARED((NPAD, D), jnp.float32),
            pltpu.SemaphoreType.DMA,
            pltpu.SemaphoreType.DMA,
        ],
    )
    def agg_kernel(hp_hbm, src_hbm, dst_hbm, zrows_hbm, out_hbm,
                   sidx, didx, rows, acc_sh, sem0, sem1):
        c = lax.axis_index("c")
        s = lax.axis_index("s")
        w = c * NS + s
        r0 = s * RPT
        sems = (sem0, sem1)

        pltpu.sync_copy(zrows_hbm, rows.at[0])
        for q0, qn in wb_chunks:
            pltpu.sync_copy(rows.at[0, pl.ds(0, qn)],
                            acc_sh.at[pl.ds(r0 + q0, qn), :])
        pltpu.sync_copy(src_hbm.at[pl.ds(w * (nrow * CH), nrow * CH)], sidx)
        pltpu.sync_copy(dst_hbm.at[pl.ds(w * nrow, nrow)], didx)
        plsc.subcore_barrier()

        def gather(j, b):
            return pltpu.make_async_copy(hp_hbm.at[sidx.at[pl.ds(j * CH, CH)]],
                                         rows.at[b], sems[b])

        gather(0, 0).start()
        gather(1, 1).start()

        def body(jj, carry):
            for b in (0, 1):
                j = jj * 2 + b

                @pl.when(j < ncht)
                def _process():
                    gather(j, b).wait()
                    pltpu.sync_copy(rows.at[b], acc_sh.at[didx.at[j]],
                                    add=True)

                    @pl.when(j + 2 < ncht)
                    def _next():
                        gather(j + 2, b).start()
            return carry

        lax.fori_loop(0, (ncht + 1) // 2, body, 0)
        plsc.subcore_barrier()

        def wb(i, phase):
            q0, qn = wb_chunks[i]
            b = i % 2
            cp = pltpu.make_async_copy(
                rows.at[b, pl.ds(0, qn)],
                out_hbm.at[pl.ds(c * NPAD + r0 + q0, qn), :], sems[b])
            if phase == 0:
                pltpu.sync_copy(acc_sh.at[pl.ds(r0 + q0, qn), :],
                                rows.at[b, pl.ds(0, qn)])
                cp.start()
            else:
                cp.wait()

        for i in range(len(wb_chunks)):
            if i >= 2:
                wb(i - 2, 1)
            wb(i, 0)
        for i in range(max(0, len(wb_chunks) - 2), len(wb_chunks)):
            wb(i, 1)

    zrows = jnp.zeros((CH, D), jnp.float32)
    return agg_kernel(hp, src_flat, dst2d, zrows)


def _tc1_body(x_ref, w_ref, d0_ref, d1_ref, out_ref, dinv_ref):
    dinv = lax.rsqrt(d0_ref[...] + d1_ref[...] + 1.0)
    dinv_ref[...] = dinv
    h = jnp.dot(x_ref[...], w_ref[...], preferred_element_type=jnp.float32)
    out_ref[...] = h * dinv


def _tc1(x, w1, deg_col):
    return pl.pallas_call(
        _tc1_body,
        grid=(NBLK,),
        in_specs=[
            pl.BlockSpec((RPB, D), lambda i: (i, 0)),
            pl.BlockSpec((D, D), lambda i: (0, 0)),
            pl.BlockSpec((RPB, 1), lambda i: (i, 0)),
            pl.BlockSpec((RPB, 1), lambda i: (i + NBLK, 0)),
        ],
        out_specs=[
            pl.BlockSpec((RPB, D), lambda i: (i, 0)),
            pl.BlockSpec((RPB, 1), lambda i: (i, 0)),
        ],
        out_shape=[
            jax.ShapeDtypeStruct((NPAD, D), jnp.float32),
            jax.ShapeDtypeStruct((NPAD, 1), jnp.float32),
        ],
    )(x, w1, deg_col, deg_col)


def _tc2_body(a0_ref, a1_ref, hp_ref, dinv_ref, b_ref, w_ref, out_ref):
    pre = dinv_ref[...] * (a0_ref[...] + a1_ref[...] + hp_ref[...]) + b_ref[...]
    x2 = jnp.maximum(pre, 0.0)
    h = jnp.dot(x2, w_ref[...], preferred_element_type=jnp.float32)
    out_ref[...] = h * dinv_ref[...]


def _tc2(g1, h1p, dinv_col, b1r, w2):
    return pl.pallas_call(
        _tc2_body,
        grid=(NBLK,),
        in_specs=[
            pl.BlockSpec((RPB, D), lambda i: (i, 0)),
            pl.BlockSpec((RPB, D), lambda i: (i + NBLK, 0)),
            pl.BlockSpec((RPB, D), lambda i: (i, 0)),
            pl.BlockSpec((RPB, 1), lambda i: (i, 0)),
            pl.BlockSpec((1, D), lambda i: (0, 0)),
            pl.BlockSpec((D, D), lambda i: (0, 0)),
        ],
        out_specs=pl.BlockSpec((RPB, D), lambda i: (i, 0)),
        out_shape=jax.ShapeDtypeStruct((NPAD, D), jnp.float32),
    )(g1, g1, h1p, dinv_col, b1r, w2)


def _tc3_body(a0_ref, a1_ref, hp_ref, dinv_ref, b_ref, out_ref):
    pre = dinv_ref[...] * (a0_ref[...] + a1_ref[...] + hp_ref[...]) + b_ref[...]
    out_ref[...] = jnp.maximum(pre, 0.0)


def _tc3(g2, h2p, dinv_col, b2r):
    return pl.pallas_call(
        _tc3_body,
        grid=(NBLK,),
        in_specs=[
            pl.BlockSpec((RPB, D), lambda i: (i, 0)),
            pl.BlockSpec((RPB, D), lambda i: (i + NBLK, 0)),
            pl.BlockSpec((RPB, D), lambda i: (i, 0)),
            pl.BlockSpec((RPB, 1), lambda i: (i, 0)),
            pl.BlockSpec((1, D), lambda i: (0, 0)),
        ],
        out_specs=pl.BlockSpec((RPB, D), lambda i: (i, 0)),
        out_shape=jax.ShapeDtypeStruct((NNODES, D), jnp.float32),
    )(g2, g2, h2p, dinv_col, b2r)


def kernel(x, edge_index, W1, b1, W2, b2):
    e = edge_index.shape[1]
    ept0 = -(-e // NTILES)
    ncht = -(-ept0 // CH)
    nrow = -(-ncht // 8) * 8
    epad = NTILES * ncht * CH
    pad = epad - e
    src = edge_index[0]
    dst = edge_index[1]
    if pad:
        ar = jnp.arange(pad, dtype=jnp.int32)
        src = jnp.concatenate([src, ar % NNODES])
        dst = jnp.concatenate([dst, NNODES + ar % (NPAD - NNODES)])

    def to_tiles(v):
        v3 = v.reshape(NTILES, ncht, CH)
        if nrow != ncht:
            v3 = jnp.pad(v3, ((0, 0), (0, nrow - ncht), (0, 0)))
        return v3.reshape(NTILES * nrow, CH)

    src_flat = to_tiles(src).reshape(-1)
    dst2d = to_tiles(dst)

    deg2 = _deg_call(dst2d, ncht, nrow)
    deg_col = deg2.reshape(NC * NPAD, 1)

    b1r = b1.reshape(1, D)
    b2r = b2.reshape(1, D)

    h1p, dinv_col = _tc1(x, W1, deg_col)
    g1 = _agg_call(h1p, src_flat, dst2d, ncht, nrow)
    h2p = _tc2(g1, h1p, dinv_col, b1r, W2)
    g2 = _agg_call(h2p, src_flat, dst2d, ncht, nrow)
    return _tc3(g2, h2p, dinv_col, b2r)
